# baseline (speedup 1.0000x reference)
import functools

import jax
import jax.numpy as jnp
from jax import lax
from jax.experimental import pallas as pl
from jax.experimental.pallas import tpu as pltpu
from jax.experimental.pallas import tpu_sc as plsc

SHAPE = (513, 513)
SCALE = 3.0
DIM = 64

NC = 2
NS = 16
L = 16
NW = NC * NS

B = 4096 * 200
C = 512
SUB = 128
NSUB = C // SUB
PER_W = B // NW
NCHUNK = PER_W // C


def _body(xq_hbm, yq_hbm, table_hbm, out_hbm,
          xbuf, ybuf, idx, rows, sem_g):
    wid = lax.axis_index("s") * NC + lax.axis_index("c")
    base = wid * PER_W

    dx = (SHAPE[0] - 1) // 2
    dy = (SHAPE[1] - 1) // 2

    def chunk(g, _):
        cbase = base + g * C
        pltpu.sync_copy(xq_hbm.at[pl.ds(cbase, C)], xbuf)
        pltpu.sync_copy(yq_hbm.at[pl.ds(cbase, C)], ybuf)

        for s in range(NSUB):
            def compute(j, _, s=s):
                off = s * SUB + j * L
                xv = xbuf[pl.ds(off, L)]
                yv = ybuf[pl.ds(off, L)]
                ix = (xv * float(dx) / SCALE + float(dx)).astype(jnp.int32)
                iy = (yv * float(dy) / SCALE + float(dy)).astype(jnp.int32)
                ix = jnp.clip(ix, 0, SHAPE[0] - 1)
                iy = jnp.clip(iy, 0, SHAPE[1] - 1)
                idx[s, pl.ds(j * L, L)] = ix * SHAPE[1] + iy
                return 0

            lax.fori_loop(0, SUB // L, compute, 0, unroll=4)

        copies = []
        for s in range(NSUB):
            copies.append(pltpu.async_copy(
                table_hbm.at[idx.at[s]],
                rows.at[pl.ds(s * SUB, SUB), :],
                sem_g))
        for cp in copies:
            cp.wait()

        pltpu.sync_copy(rows, out_hbm.at[pl.ds(cbase, C)])
        return 0

    lax.fori_loop(0, NCHUNK, chunk, 0)


@jax.jit
def _xy_gather(xq, yq, table):
    mesh = plsc.VectorSubcoreMesh(core_axis_name="c", subcore_axis_name="s")
    f = pl.kernel(
        _body,
        out_type=jax.ShapeDtypeStruct((B, DIM), jnp.float32),
        mesh=mesh,
        scratch_types=[
            pltpu.VMEM((C,), jnp.float32),
            pltpu.VMEM((C,), jnp.float32),
            pltpu.VMEM((NSUB, SUB), jnp.int32),
            pltpu.VMEM((C, DIM), jnp.float32),
            pltpu.SemaphoreType.DMA,
        ],
        compiler_params=pltpu.CompilerParams(use_tc_tiling_on_sc=False),
    )
    return f(xq, yq, table)


def kernel(pos, embedding):
    n, t, _ = pos.shape
    xq = pos[..., 0].reshape(-1)
    yq = pos[..., 1].reshape(-1)
    table = embedding.reshape(SHAPE[0] * SHAPE[1], DIM)
    out = _xy_gather(xq, yq, table)
    return out.reshape(n, t, DIM)
